# Initial kernel scaffold; baseline (speedup 1.0000x reference)
import jax
import jax.numpy as jnp
from jax.experimental import pallas as pl


def kernel(hidden_states, gate_weight, W_gate, W_up, W_down, Ws_gate, Ws_up, Ws_down):
    raise NotImplementedError("write your pallas kernel here")


# speedup vs baseline: 2.1838x; 2.1838x over previous
import functools

import jax
import jax.numpy as jnp
from jax.experimental import pallas as pl
from jax.experimental.pallas import tpu as pltpu

E = 8
TOP_K = 2


def _moe_body(x_ref, gw_ref, wg_ref, wu_ref, wd_ref, sg_ref, su_ref, sd_ref,
              out_ref):
    x = x_ref[...]

    logits = jax.lax.dot_general(
        x, gw_ref[...], (((1,), (1,)), ((), ())),
        preferred_element_type=jnp.float32)
    m1 = jnp.max(logits, axis=1, keepdims=True)
    masked = jnp.where(logits < m1, logits, -jnp.inf)
    m2 = jnp.max(masked, axis=1, keepdims=True)
    denom = 1.0 + jnp.exp(m2 - m1)
    wmat = jnp.where(logits >= m2, jnp.exp(logits - m1) / denom, 0.0)

    acc = jnp.zeros(out_ref.shape, jnp.float32)
    for e in range(E):
        g = jnp.dot(x, wg_ref[e], preferred_element_type=jnp.float32)
        u = jnp.dot(x, wu_ref[e], preferred_element_type=jnp.float32)
        h = g * jax.nn.sigmoid(g) * u
        d = jnp.dot(h, wd_ref[e], preferred_element_type=jnp.float32)
        acc = acc + wmat[:, e:e + 1] * d

    sg = jnp.dot(x, sg_ref[...], preferred_element_type=jnp.float32)
    su = jnp.dot(x, su_ref[...], preferred_element_type=jnp.float32)
    sh = sg * jax.nn.sigmoid(sg) * su
    acc = acc + jnp.dot(sh, sd_ref[...], preferred_element_type=jnp.float32)

    out_ref[...] = acc


def kernel(hidden_states, gate_weight, W_gate, W_up, W_down, Ws_gate, Ws_up,
           Ws_down):
    orig_shape = hidden_states.shape
    D = orig_shape[-1]
    x = hidden_states.reshape(-1, D)
    T = x.shape[0]
    TB = 512
    F = W_gate.shape[-1]
    SF = Ws_gate.shape[-1]

    full = lambda *shape: pl.BlockSpec(shape, lambda i: (0,) * len(shape))
    out = pl.pallas_call(
        _moe_body,
        grid=(T // TB,),
        in_specs=[
            pl.BlockSpec((TB, D), lambda i: (i, 0)),
            full(E, D),
            full(E, D, F),
            full(E, D, F),
            full(E, F, D),
            full(D, SF),
            full(D, SF),
            full(SF, D),
        ],
        out_specs=pl.BlockSpec((TB, D), lambda i: (i, 0)),
        out_shape=jax.ShapeDtypeStruct((T, D), jnp.float32),
        compiler_params=pltpu.CompilerParams(
            dimension_semantics=("arbitrary",),
            vmem_limit_bytes=110 * 1024 * 1024,
        ),
    )(x, gate_weight, W_gate, W_up, W_down, Ws_gate, Ws_up, Ws_down)
    return out.reshape(orig_shape)


# speedup vs baseline: 2.2009x; 1.0078x over previous
import functools

import jax
import jax.numpy as jnp
from jax.experimental import pallas as pl
from jax.experimental.pallas import tpu as pltpu

E = 8
TOP_K = 2


def _moe_body(x_ref, gw_ref, wg_ref, wu_ref, wd_ref, sg_ref, su_ref, sd_ref,
              out_ref):
    x = x_ref[...]

    logits = jax.lax.dot_general(
        x, gw_ref[...], (((1,), (1,)), ((), ())),
        preferred_element_type=jnp.float32)
    m1 = jnp.max(logits, axis=1, keepdims=True)
    masked = jnp.where(logits < m1, logits, -jnp.inf)
    m2 = jnp.max(masked, axis=1, keepdims=True)
    denom = 1.0 + jnp.exp(m2 - m1)
    wmat = jnp.where(logits >= m2, jnp.exp(logits - m1) / denom, 0.0)

    xb = x.astype(jnp.bfloat16)
    acc = jnp.zeros(out_ref.shape, jnp.float32)
    for e in range(E):
        g = jnp.dot(xb, wg_ref[e].astype(jnp.bfloat16),
                    preferred_element_type=jnp.float32)
        u = jnp.dot(xb, wu_ref[e].astype(jnp.bfloat16),
                    preferred_element_type=jnp.float32)
        h = g * jax.nn.sigmoid(g) * u
        d = jnp.dot(h.astype(jnp.bfloat16), wd_ref[e].astype(jnp.bfloat16),
                    preferred_element_type=jnp.float32)
        acc = acc + wmat[:, e:e + 1] * d

    sg = jnp.dot(xb, sg_ref[...].astype(jnp.bfloat16),
                 preferred_element_type=jnp.float32)
    su = jnp.dot(xb, su_ref[...].astype(jnp.bfloat16),
                 preferred_element_type=jnp.float32)
    sh = sg * jax.nn.sigmoid(sg) * su
    acc = acc + jnp.dot(sh.astype(jnp.bfloat16), sd_ref[...].astype(jnp.bfloat16),
                        preferred_element_type=jnp.float32)

    out_ref[...] = acc


def kernel(hidden_states, gate_weight, W_gate, W_up, W_down, Ws_gate, Ws_up,
           Ws_down):
    orig_shape = hidden_states.shape
    D = orig_shape[-1]
    x = hidden_states.reshape(-1, D)
    T = x.shape[0]
    TB = 512
    F = W_gate.shape[-1]
    SF = Ws_gate.shape[-1]

    full = lambda *shape: pl.BlockSpec(shape, lambda i: (0,) * len(shape))
    out = pl.pallas_call(
        _moe_body,
        grid=(T // TB,),
        in_specs=[
            pl.BlockSpec((TB, D), lambda i: (i, 0)),
            full(E, D),
            full(E, D, F),
            full(E, D, F),
            full(E, F, D),
            full(D, SF),
            full(D, SF),
            full(SF, D),
        ],
        out_specs=pl.BlockSpec((TB, D), lambda i: (i, 0)),
        out_shape=jax.ShapeDtypeStruct((T, D), jnp.float32),
        compiler_params=pltpu.CompilerParams(
            dimension_semantics=("arbitrary",),
            vmem_limit_bytes=110 * 1024 * 1024,
        ),
    )(x, gate_weight, W_gate, W_up, W_down, Ws_gate, Ws_up, Ws_down)
    return out.reshape(orig_shape)
